# baseline (speedup 1.0000x reference)
import jax
import jax.numpy as jnp
from jax.experimental import pallas as pl
from jax.experimental.pallas import tpu as pltpu

R_MAX, S_MAX = 32, 2
W_RES = 2 * R_MAX + 2
W_CH = 2 * S_MAX + 2
D_RELPOS = 2 * W_RES + 1 + W_CH

_TI = 64


def _body(ai_ref, w_atom_ref, lens_ref, atf_ref, w_token_ref, ws_a_ref,
          ws_b_ref, molid_ref, mte_ref, modm_ref, mode_ref, wpi_a_ref,
          wpi_b_ref, wpj_a_ref, wpj_b_ref, meta_i_ref, metaT_ref, tbr_ref,
          tbc_ref, w_relpos_ref, w_bond_ref, single_ref, out_ref,
          pi_s, pj_s):
    f32 = jnp.float32
    bf = jnp.bfloat16
    step = pl.program_id(0)

    @pl.when(step == 0)
    def _prologue():
        af = jnp.dot(ai_ref[...].astype(bf), w_atom_ref[...].astype(bf),
                     preferred_element_type=f32).astype(bf)
        n = lens_ref.shape[0]
        m = af.shape[0]
        lens = lens_ref[...]
        r = jax.lax.broadcasted_iota(jnp.int32, (n, n), 0)
        c = jax.lax.broadcasted_iota(jnp.int32, (n, n), 1)
        tri = (r >= c).astype(f32)
        csum = jnp.dot(tri, lens, preferred_element_type=f32)
        start = csum - lens
        aio = jax.lax.broadcasted_iota(jnp.int32, (n, m), 1).astype(f32)
        assign = ((aio >= start) & (aio < csum)).astype(bf)
        summed = jnp.dot(assign, af, preferred_element_type=f32)
        pooled = summed / jnp.maximum(lens, 1.0)
        tokfeat = jnp.dot(pooled, w_token_ref[...],
                          preferred_element_type=f32)
        atf = atf_ref[...]
        molid = molid_ref[...]
        moh = (molid == jax.lax.broadcasted_iota(
            jnp.int32, (n, mte_ref.shape[0]), 1)).astype(f32)
        single_ref[...] = (
            jnp.dot(tokfeat, ws_a_ref[...], preferred_element_type=f32)
            + jnp.dot(atf, ws_b_ref[...], preferred_element_type=f32)
            + jnp.dot(moh, mte_ref[...], preferred_element_type=f32)
            + jnp.dot(modm_ref[...], mode_ref[...],
                      preferred_element_type=f32))
        pi_s[...] = (
            jnp.dot(tokfeat, wpi_a_ref[...], preferred_element_type=f32)
            + jnp.dot(atf, wpi_b_ref[...], preferred_element_type=f32))
        pj_s[...] = (
            jnp.dot(tokfeat, wpj_a_ref[...], preferred_element_type=f32)
            + jnp.dot(atf, wpj_b_ref[...], preferred_element_type=f32))

    ti = out_ref.shape[0]
    n = pj_s.shape[0]
    mi = meta_i_ref[...]
    mj = metaT_ref[...]
    res_i, tok_i, asym_i, ent_i, sym_i, bio_i = (
        mi[:, k:k + 1] for k in range(6))
    res_j, tok_j, asym_j, ent_j, sym_j, bio_j = (
        mj[k:k + 1, :] for k in range(6))
    same_chain = asym_i == asym_j
    same_entity = ent_i == ent_j
    d_res = jnp.where(same_chain,
                      jnp.clip(res_i - res_j + R_MAX, 0, 2 * R_MAX),
                      2 * R_MAX + 1)
    d_tok = jnp.where(same_chain,
                      jnp.clip(tok_i - tok_j + R_MAX, 0, 2 * R_MAX),
                      2 * R_MAX + 1)
    d_chain = jnp.where(same_entity,
                        jnp.clip(sym_i - sym_j + S_MAX, 0, 2 * S_MAX),
                        2 * S_MAX + 1)
    bio = (bio_i > 0) & (bio_j > 0)
    neg = jnp.full((), -1, jnp.int32)
    d_res = jnp.where(bio, d_res, neg)
    d_tok = jnp.where(bio, d_tok, neg)
    d_chain = jnp.where(bio, d_chain, neg)
    seb_i = jnp.where(same_entity & bio, 1, 0)
    c72 = jax.lax.broadcasted_iota(jnp.int32, (ti, 72, n), 1)
    c8 = jax.lax.broadcasted_iota(jnp.int32, (ti, 8, n), 1)
    ohr = (c72 == d_res.reshape(ti, 1, n)).astype(bf)
    oht = (c72 == d_tok.reshape(ti, 1, n)).astype(bf)
    sebt = (c8 == 0).astype(bf) * seb_i.reshape(ti, 1, n).astype(bf)
    ohc = (c8 == d_chain.reshape(ti, 1, n)).astype(bf)
    i0 = step * ti
    csel = jax.lax.broadcasted_iota(jnp.int32, (ti, ti, n), 1)
    rsel = jax.lax.broadcasted_iota(jnp.int32, (ti, ti, n), 0)
    pio = (csel == rsel).astype(bf)
    tb_tile = jnp.maximum(tbr_ref[...], tbc_ref[...])
    rk = jax.lax.broadcasted_iota(jnp.int32, (ti, n), 0)
    cj = jax.lax.broadcasted_iota(jnp.int32, (ti, n), 1)
    tb_tile = jnp.where(cj == i0 + rk, 0.0, tb_tile)
    tb8 = jnp.concatenate(
        [tb_tile.reshape(ti, 1, n).astype(bf),
         jnp.zeros((ti, 7, n), bf)], axis=1)
    mh_t = jnp.concatenate([ohr, oht, sebt, ohc, pio, tb8],
                           axis=1)
    pjv = pj_s[...]
    dpw = pjv.shape[1]
    pi_tile = pi_s[pl.ds(i0, ti), :]
    w_full = jnp.concatenate(
        [w_relpos_ref[...], pi_tile.astype(bf),
         w_bond_ref[...].astype(bf),
         jnp.zeros((8 - 1, dpw), bf)], axis=0)
    for k in range(ti):
        relk = jax.lax.dot_general(
            mh_t[k], w_full, (((0,), (0,)), ((), ())),
            preferred_element_type=f32)
        out_ref[k] = relk + pjv


def kernel(atom_inputs, atom_ids, atom_mask, atompair_inputs, atompair_ids,
           valid_atom_indices_for_frame, token_bonds, additional_token_feats,
           molecule_atom_lens, molecule_atom_indices, molecule_ids,
           is_molecule_mod, is_molecule_types, additional_molecule_feats,
           distogram_atom_indices, atom_indices_for_frame, W_atom, W_atompair,
           W_token, W_single, mol_type_emb, W_pair_i, W_pair_j,
           atom_emb_table, atompair_emb_table, mod_emb_table, W_relpos,
           w_bond):
    f32 = jnp.float32
    n = molecule_atom_lens.shape[1]
    m = atom_inputs.shape[1]
    dp = W_pair_i.shape[1]
    ds = W_single.shape[1]
    dtok = W_token.shape[1]
    da_in = atom_inputs.shape[2]
    da = W_atom.shape[1]
    dadd = additional_token_feats.shape[2]
    nmol = mol_type_emb.shape[0]

    ai = atom_inputs[0]
    lens_i = jnp.maximum(molecule_atom_lens[0].astype(jnp.int32), 0)
    lens_col = lens_i[:, None].astype(f32)
    atf = additional_token_feats[0]
    molid = molecule_ids[0][:, None].astype(jnp.int32)
    modm = jnp.pad(is_molecule_mod[0].astype(f32), ((0, 0), (0, 4)))
    mode = jnp.pad(mod_emb_table, ((0, 4), (0, 0)))
    ws_a, ws_b = W_single[:dtok], W_single[dtok:]
    wpi_a, wpi_b = W_pair_i[:dtok], W_pair_i[dtok:]
    wpj_a, wpj_b = W_pair_j[:dtok], W_pair_j[dtok:]

    amf = additional_molecule_feats[0].astype(jnp.int32)
    biomol = is_molecule_types[0, :, 0:3].any(axis=-1)
    meta = jnp.concatenate(
        [amf, biomol[:, None].astype(jnp.int32),
         jnp.zeros((n, 2), jnp.int32)], axis=1)
    metaT = meta.T
    tbf = token_bonds[0].astype(f32)
    tbfT = tbf.T
    zcol = jnp.zeros((1, dp), f32)
    w_big = jnp.concatenate([
        W_relpos[0:W_RES], jnp.tile(zcol, (72 - W_RES, 1)),
        W_relpos[W_RES:2 * W_RES], jnp.tile(zcol, (72 - W_RES, 1)),
        W_relpos[2 * W_RES:2 * W_RES + 1], jnp.tile(zcol, (7, 1)),
        W_relpos[2 * W_RES + 1:], jnp.tile(zcol, (8 - W_CH, 1)),
    ], axis=0).astype(jnp.bfloat16)

    cst = lambda *_: tuple(0 for _ in range(2))
    single_init, pairwise = pl.pallas_call(
        _body,
        grid=(n // _TI,),
        in_specs=[
            pl.BlockSpec((m, da_in), lambda i: (0, 0)),
            pl.BlockSpec((da_in, da), lambda i: (0, 0)),
            pl.BlockSpec((n, 1), lambda i: (0, 0)),
            pl.BlockSpec((n, dadd), lambda i: (0, 0)),
            pl.BlockSpec((da, dtok), lambda i: (0, 0)),
            pl.BlockSpec((dtok, ds), lambda i: (0, 0)),
            pl.BlockSpec((dadd, ds), lambda i: (0, 0)),
            pl.BlockSpec((n, 1), lambda i: (0, 0)),
            pl.BlockSpec((nmol, ds), lambda i: (0, 0)),
            pl.BlockSpec((n, 8), lambda i: (0, 0)),
            pl.BlockSpec((8, ds), lambda i: (0, 0)),
            pl.BlockSpec((dtok, dp), lambda i: (0, 0)),
            pl.BlockSpec((dadd, dp), lambda i: (0, 0)),
            pl.BlockSpec((dtok, dp), lambda i: (0, 0)),
            pl.BlockSpec((dadd, dp), lambda i: (0, 0)),
            pl.BlockSpec((_TI, 8), lambda i: (i, 0)),
            pl.BlockSpec((8, n), lambda i: (0, 0)),
            pl.BlockSpec((_TI, n), lambda i: (i, 0)),
            pl.BlockSpec((_TI, n), lambda i: (i, 0)),
            pl.BlockSpec((160, dp), lambda i: (0, 0)),
            pl.BlockSpec((1, dp), lambda i: (0, 0)),
        ],
        out_specs=(
            pl.BlockSpec((n, ds), lambda i: (0, 0)),
            pl.BlockSpec((_TI, n, dp), lambda i: (i, 0, 0)),
        ),
        out_shape=(jax.ShapeDtypeStruct((n, ds), f32),
                   jax.ShapeDtypeStruct((n, n, dp), f32)),
        scratch_shapes=[pltpu.VMEM((n, dp), f32),
                        pltpu.VMEM((n, dp), f32)],
    )(ai, W_atom, lens_col, atf, W_token, ws_a, ws_b, molid, mol_type_emb,
      modm, mode, wpi_a, wpi_b, wpj_a, wpj_b, meta, metaT, tbf, tbfT,
      w_big, w_bond)

    single_mask = (lens_i > 0)[None]
    pairwise_mask = single_mask[:, :, None] & single_mask[:, None, :]
    return single_init[None], single_mask, pairwise[None], pairwise_mask


# speedup vs baseline: 1.0042x; 1.0042x over previous
import jax
import jax.numpy as jnp
from jax.experimental import pallas as pl
from jax.experimental.pallas import tpu as pltpu

R_MAX, S_MAX = 32, 2
W_RES = 2 * R_MAX + 2
W_CH = 2 * S_MAX + 2
D_RELPOS = 2 * W_RES + 1 + W_CH

_TI = 32


def _body(ai_ref, w_atom_ref, lens_ref, atf_ref, w_token_ref, ws_a_ref,
          ws_b_ref, molid_ref, mte_ref, modm_ref, mode_ref, wpi_a_ref,
          wpi_b_ref, wpj_a_ref, wpj_b_ref, meta_i_ref, metaT_ref, tbr_ref,
          tbc_ref, w_relpos_ref, w_bond_ref, single_ref, out_ref,
          pi_s, pj_s):
    f32 = jnp.float32
    bf = jnp.bfloat16
    step = pl.program_id(0)

    @pl.when(step == 0)
    def _prologue():
        af = jnp.dot(ai_ref[...].astype(bf), w_atom_ref[...].astype(bf),
                     preferred_element_type=f32).astype(bf)
        n = lens_ref.shape[0]
        m = af.shape[0]
        lens = lens_ref[...]
        r = jax.lax.broadcasted_iota(jnp.int32, (n, n), 0)
        c = jax.lax.broadcasted_iota(jnp.int32, (n, n), 1)
        tri = (r >= c).astype(f32)
        csum = jnp.dot(tri, lens, preferred_element_type=f32)
        start = csum - lens
        aio = jax.lax.broadcasted_iota(jnp.int32, (n, m), 1).astype(f32)
        assign = ((aio >= start) & (aio < csum)).astype(bf)
        summed = jnp.dot(assign, af, preferred_element_type=f32)
        pooled = summed / jnp.maximum(lens, 1.0)
        tokfeat = jnp.dot(pooled, w_token_ref[...],
                          preferred_element_type=f32)
        atf = atf_ref[...]
        molid = molid_ref[...]
        moh = (molid == jax.lax.broadcasted_iota(
            jnp.int32, (n, mte_ref.shape[0]), 1)).astype(f32)
        single_ref[...] = (
            jnp.dot(tokfeat, ws_a_ref[...], preferred_element_type=f32)
            + jnp.dot(atf, ws_b_ref[...], preferred_element_type=f32)
            + jnp.dot(moh, mte_ref[...], preferred_element_type=f32)
            + jnp.dot(modm_ref[...], mode_ref[...],
                      preferred_element_type=f32))
        pi_s[...] = (
            jnp.dot(tokfeat, wpi_a_ref[...], preferred_element_type=f32)
            + jnp.dot(atf, wpi_b_ref[...], preferred_element_type=f32))
        pj_s[...] = (
            jnp.dot(tokfeat, wpj_a_ref[...], preferred_element_type=f32)
            + jnp.dot(atf, wpj_b_ref[...], preferred_element_type=f32))

    ti = out_ref.shape[0]
    n = pj_s.shape[0]
    mi = meta_i_ref[...]
    mj = metaT_ref[...]
    res_i, tok_i, asym_i, ent_i, sym_i, bio_i = (
        mi[:, k:k + 1] for k in range(6))
    res_j, tok_j, asym_j, ent_j, sym_j, bio_j = (
        mj[k:k + 1, :] for k in range(6))
    same_chain = asym_i == asym_j
    same_entity = ent_i == ent_j
    d_res = jnp.where(same_chain,
                      jnp.clip(res_i - res_j + R_MAX, 0, 2 * R_MAX),
                      2 * R_MAX + 1)
    d_tok = jnp.where(same_chain,
                      jnp.clip(tok_i - tok_j + R_MAX, 0, 2 * R_MAX),
                      2 * R_MAX + 1)
    d_chain = jnp.where(same_entity,
                        jnp.clip(sym_i - sym_j + S_MAX, 0, 2 * S_MAX),
                        2 * S_MAX + 1)
    bio = (bio_i > 0) & (bio_j > 0)
    neg = jnp.full((), -1, jnp.int32)
    d_res = jnp.where(bio, d_res, neg)
    d_tok = jnp.where(bio, d_tok, neg)
    d_chain = jnp.where(bio, d_chain, neg)
    seb_i = jnp.where(same_entity & bio, 1, 0)
    c72 = jax.lax.broadcasted_iota(jnp.int32, (ti, 72, n), 1)
    c8 = jax.lax.broadcasted_iota(jnp.int32, (ti, 8, n), 1)
    ohr = (c72 == d_res.reshape(ti, 1, n)).astype(bf)
    oht = (c72 == d_tok.reshape(ti, 1, n)).astype(bf)
    sebt = (c8 == 0).astype(bf) * seb_i.reshape(ti, 1, n).astype(bf)
    ohc = (c8 == d_chain.reshape(ti, 1, n)).astype(bf)
    i0 = step * ti
    csel = jax.lax.broadcasted_iota(jnp.int32, (ti, ti, n), 1)
    rsel = jax.lax.broadcasted_iota(jnp.int32, (ti, ti, n), 0)
    pio = (csel == rsel).astype(bf)
    tb_tile = jnp.maximum(tbr_ref[...], tbc_ref[...])
    rk = jax.lax.broadcasted_iota(jnp.int32, (ti, n), 0)
    cj = jax.lax.broadcasted_iota(jnp.int32, (ti, n), 1)
    tb_tile = jnp.where(cj == i0 + rk, 0.0, tb_tile)
    tb8 = jnp.concatenate(
        [tb_tile.reshape(ti, 1, n).astype(bf),
         jnp.zeros((ti, 7, n), bf)], axis=1)
    mh_t = jnp.concatenate([ohr, oht, sebt, ohc, pio, tb8],
                           axis=1)
    pjv = pj_s[...]
    dpw = pjv.shape[1]
    pi_tile = pi_s[pl.ds(i0, ti), :]
    w_full = jnp.concatenate(
        [w_relpos_ref[...], pi_tile.astype(bf),
         w_bond_ref[...].astype(bf),
         jnp.zeros((8 - 1, dpw), bf)], axis=0)
    for k in range(ti):
        relk = jax.lax.dot_general(
            mh_t[k], w_full, (((0,), (0,)), ((), ())),
            preferred_element_type=f32)
        out_ref[k] = relk + pjv


def kernel(atom_inputs, atom_ids, atom_mask, atompair_inputs, atompair_ids,
           valid_atom_indices_for_frame, token_bonds, additional_token_feats,
           molecule_atom_lens, molecule_atom_indices, molecule_ids,
           is_molecule_mod, is_molecule_types, additional_molecule_feats,
           distogram_atom_indices, atom_indices_for_frame, W_atom, W_atompair,
           W_token, W_single, mol_type_emb, W_pair_i, W_pair_j,
           atom_emb_table, atompair_emb_table, mod_emb_table, W_relpos,
           w_bond):
    f32 = jnp.float32
    n = molecule_atom_lens.shape[1]
    m = atom_inputs.shape[1]
    dp = W_pair_i.shape[1]
    ds = W_single.shape[1]
    dtok = W_token.shape[1]
    da_in = atom_inputs.shape[2]
    da = W_atom.shape[1]
    dadd = additional_token_feats.shape[2]
    nmol = mol_type_emb.shape[0]

    ai = atom_inputs[0]
    lens_i = jnp.maximum(molecule_atom_lens[0].astype(jnp.int32), 0)
    lens_col = lens_i[:, None].astype(f32)
    atf = additional_token_feats[0]
    molid = molecule_ids[0][:, None].astype(jnp.int32)
    modm = jnp.pad(is_molecule_mod[0].astype(f32), ((0, 0), (0, 4)))
    mode = jnp.pad(mod_emb_table, ((0, 4), (0, 0)))
    ws_a, ws_b = W_single[:dtok], W_single[dtok:]
    wpi_a, wpi_b = W_pair_i[:dtok], W_pair_i[dtok:]
    wpj_a, wpj_b = W_pair_j[:dtok], W_pair_j[dtok:]

    amf = additional_molecule_feats[0].astype(jnp.int32)
    biomol = is_molecule_types[0, :, 0:3].any(axis=-1)
    meta = jnp.concatenate(
        [amf, biomol[:, None].astype(jnp.int32),
         jnp.zeros((n, 2), jnp.int32)], axis=1)
    metaT = meta.T
    tbf = token_bonds[0].astype(f32)
    tbfT = tbf.T
    zcol = jnp.zeros((1, dp), f32)
    w_big = jnp.concatenate([
        W_relpos[0:W_RES], jnp.tile(zcol, (72 - W_RES, 1)),
        W_relpos[W_RES:2 * W_RES], jnp.tile(zcol, (72 - W_RES, 1)),
        W_relpos[2 * W_RES:2 * W_RES + 1], jnp.tile(zcol, (7, 1)),
        W_relpos[2 * W_RES + 1:], jnp.tile(zcol, (8 - W_CH, 1)),
    ], axis=0).astype(jnp.bfloat16)

    cst = lambda *_: tuple(0 for _ in range(2))
    single_init, pairwise = pl.pallas_call(
        _body,
        grid=(n // _TI,),
        in_specs=[
            pl.BlockSpec((m, da_in), lambda i: (0, 0)),
            pl.BlockSpec((da_in, da), lambda i: (0, 0)),
            pl.BlockSpec((n, 1), lambda i: (0, 0)),
            pl.BlockSpec((n, dadd), lambda i: (0, 0)),
            pl.BlockSpec((da, dtok), lambda i: (0, 0)),
            pl.BlockSpec((dtok, ds), lambda i: (0, 0)),
            pl.BlockSpec((dadd, ds), lambda i: (0, 0)),
            pl.BlockSpec((n, 1), lambda i: (0, 0)),
            pl.BlockSpec((nmol, ds), lambda i: (0, 0)),
            pl.BlockSpec((n, 8), lambda i: (0, 0)),
            pl.BlockSpec((8, ds), lambda i: (0, 0)),
            pl.BlockSpec((dtok, dp), lambda i: (0, 0)),
            pl.BlockSpec((dadd, dp), lambda i: (0, 0)),
            pl.BlockSpec((dtok, dp), lambda i: (0, 0)),
            pl.BlockSpec((dadd, dp), lambda i: (0, 0)),
            pl.BlockSpec((_TI, 8), lambda i: (i, 0)),
            pl.BlockSpec((8, n), lambda i: (0, 0)),
            pl.BlockSpec((_TI, n), lambda i: (i, 0)),
            pl.BlockSpec((_TI, n), lambda i: (i, 0)),
            pl.BlockSpec((160, dp), lambda i: (0, 0)),
            pl.BlockSpec((1, dp), lambda i: (0, 0)),
        ],
        out_specs=(
            pl.BlockSpec((n, ds), lambda i: (0, 0)),
            pl.BlockSpec((_TI, n, dp), lambda i: (i, 0, 0)),
        ),
        out_shape=(jax.ShapeDtypeStruct((n, ds), f32),
                   jax.ShapeDtypeStruct((n, n, dp), f32)),
        scratch_shapes=[pltpu.VMEM((n, dp), f32),
                        pltpu.VMEM((n, dp), f32)],
    )(ai, W_atom, lens_col, atf, W_token, ws_a, ws_b, molid, mol_type_emb,
      modm, mode, wpi_a, wpi_b, wpj_a, wpj_b, meta, metaT, tbf, tbfT,
      w_big, w_bond)

    single_mask = (lens_i > 0)[None]
    pairwise_mask = single_mask[:, :, None] & single_mask[:, None, :]
    return single_init[None], single_mask, pairwise[None], pairwise_mask


# speedup vs baseline: 1.0116x; 1.0074x over previous
import jax
import jax.numpy as jnp
from jax.experimental import pallas as pl
from jax.experimental.pallas import tpu as pltpu

R_MAX, S_MAX = 32, 2
W_RES = 2 * R_MAX + 2
W_CH = 2 * S_MAX + 2
D_RELPOS = 2 * W_RES + 1 + W_CH

_TI = 32


def _body(ai_ref, w_atom_ref, lens_ref, atf_ref, w_token_ref, ws_a_ref,
          ws_b_ref, molid_ref, mte_ref, modm_ref, mode_ref, wpi_a_ref,
          wpi_b_ref, wpj_a_ref, wpj_b_ref, meta_i_ref, metaT_ref, tbr_ref,
          tbc_ref, w_relpos_ref, w_bond_ref, single_ref, out_ref,
          tokfeat_s, pj_s):
    f32 = jnp.float32
    bf = jnp.bfloat16
    step = pl.program_id(0)

    @pl.when(step == 0)
    def _prologue():
        af = jnp.dot(ai_ref[...].astype(bf), w_atom_ref[...].astype(bf),
                     preferred_element_type=f32).astype(bf)
        n = lens_ref.shape[0]
        m = af.shape[0]
        lens = lens_ref[...]
        r = jax.lax.broadcasted_iota(jnp.int32, (n, n), 0)
        c = jax.lax.broadcasted_iota(jnp.int32, (n, n), 1)
        tri = (r >= c).astype(f32)
        csum = jnp.dot(tri, lens, preferred_element_type=f32)
        start = csum - lens
        aio = jax.lax.broadcasted_iota(jnp.int32, (n, m), 1).astype(f32)
        assign = ((aio >= start) & (aio < csum)).astype(bf)
        summed = jnp.dot(assign, af, preferred_element_type=f32)
        pooled = summed / jnp.maximum(lens, 1.0)
        tokfeat = jnp.dot(pooled, w_token_ref[...],
                          preferred_element_type=f32)
        tokfeat_s[...] = tokfeat
        pj_s[...] = (
            jnp.dot(tokfeat, wpj_a_ref[...], preferred_element_type=f32)
            + jnp.dot(atf_ref[...], wpj_b_ref[...],
                      preferred_element_type=f32))

    @pl.when(step == 1)
    def _single():
        n = lens_ref.shape[0]
        molid = molid_ref[...]
        moh = (molid == jax.lax.broadcasted_iota(
            jnp.int32, (n, mte_ref.shape[0]), 1)).astype(f32)
        single_ref[...] = (
            jnp.dot(tokfeat_s[...], ws_a_ref[...],
                    preferred_element_type=f32)
            + jnp.dot(atf_ref[...], ws_b_ref[...],
                      preferred_element_type=f32)
            + jnp.dot(moh, mte_ref[...], preferred_element_type=f32)
            + jnp.dot(modm_ref[...], mode_ref[...],
                      preferred_element_type=f32))

    ti = out_ref.shape[0]
    n = pj_s.shape[0]
    mi = meta_i_ref[...]
    mj = metaT_ref[...]
    res_i, tok_i, asym_i, ent_i, sym_i, bio_i = (
        mi[:, k:k + 1] for k in range(6))
    res_j, tok_j, asym_j, ent_j, sym_j, bio_j = (
        mj[k:k + 1, :] for k in range(6))
    same_chain = asym_i == asym_j
    same_entity = ent_i == ent_j
    d_res = jnp.where(same_chain,
                      jnp.clip(res_i - res_j + R_MAX, 0, 2 * R_MAX),
                      2 * R_MAX + 1)
    d_tok = jnp.where(same_chain,
                      jnp.clip(tok_i - tok_j + R_MAX, 0, 2 * R_MAX),
                      2 * R_MAX + 1)
    d_chain = jnp.where(same_entity,
                        jnp.clip(sym_i - sym_j + S_MAX, 0, 2 * S_MAX),
                        2 * S_MAX + 1)
    bio = (bio_i > 0) & (bio_j > 0)
    neg = jnp.full((), -1, jnp.int32)
    d_res = jnp.where(bio, d_res, neg)
    d_tok = jnp.where(bio, d_tok, neg)
    d_chain = jnp.where(bio, d_chain, neg)
    seb_i = jnp.where(same_entity & bio, 1, 0)
    c72 = jax.lax.broadcasted_iota(jnp.int32, (ti, 72, n), 1)
    c8 = jax.lax.broadcasted_iota(jnp.int32, (ti, 8, n), 1)
    ohr = (c72 == d_res.reshape(ti, 1, n)).astype(bf)
    oht = (c72 == d_tok.reshape(ti, 1, n)).astype(bf)
    sebt = (c8 == 0).astype(bf) * seb_i.reshape(ti, 1, n).astype(bf)
    ohc = (c8 == d_chain.reshape(ti, 1, n)).astype(bf)
    i0 = step * ti
    csel = jax.lax.broadcasted_iota(jnp.int32, (ti, ti, n), 1)
    rsel = jax.lax.broadcasted_iota(jnp.int32, (ti, ti, n), 0)
    pio = (csel == rsel).astype(bf)
    tb_tile = jnp.maximum(tbr_ref[...], tbc_ref[...])
    rk = jax.lax.broadcasted_iota(jnp.int32, (ti, n), 0)
    cj = jax.lax.broadcasted_iota(jnp.int32, (ti, n), 1)
    tb_tile = jnp.where(cj == i0 + rk, 0.0, tb_tile)
    tb8 = jnp.concatenate(
        [tb_tile.reshape(ti, 1, n).astype(bf),
         jnp.zeros((ti, 7, n), bf)], axis=1)
    mh_t = jnp.concatenate([ohr, oht, sebt, ohc, pio, tb8],
                           axis=1)
    pjv = pj_s[...]
    dpw = pjv.shape[1]
    pi_tile = (
        jnp.dot(tokfeat_s[pl.ds(i0, ti), :], wpi_a_ref[...],
                preferred_element_type=f32)
        + jnp.dot(atf_ref[pl.ds(i0, ti), :], wpi_b_ref[...],
                  preferred_element_type=f32))
    w_full = jnp.concatenate(
        [w_relpos_ref[...], pi_tile.astype(bf),
         w_bond_ref[...].astype(bf),
         jnp.zeros((8 - 1, dpw), bf)], axis=0)
    for k in range(ti):
        relk = jax.lax.dot_general(
            mh_t[k], w_full, (((0,), (0,)), ((), ())),
            preferred_element_type=f32)
        out_ref[k] = relk + pjv


def kernel(atom_inputs, atom_ids, atom_mask, atompair_inputs, atompair_ids,
           valid_atom_indices_for_frame, token_bonds, additional_token_feats,
           molecule_atom_lens, molecule_atom_indices, molecule_ids,
           is_molecule_mod, is_molecule_types, additional_molecule_feats,
           distogram_atom_indices, atom_indices_for_frame, W_atom, W_atompair,
           W_token, W_single, mol_type_emb, W_pair_i, W_pair_j,
           atom_emb_table, atompair_emb_table, mod_emb_table, W_relpos,
           w_bond):
    f32 = jnp.float32
    n = molecule_atom_lens.shape[1]
    m = atom_inputs.shape[1]
    dp = W_pair_i.shape[1]
    ds = W_single.shape[1]
    dtok = W_token.shape[1]
    da_in = atom_inputs.shape[2]
    da = W_atom.shape[1]
    dadd = additional_token_feats.shape[2]
    nmol = mol_type_emb.shape[0]

    ai = atom_inputs[0]
    lens_i = jnp.maximum(molecule_atom_lens[0].astype(jnp.int32), 0)
    lens_col = lens_i[:, None].astype(f32)
    atf = additional_token_feats[0]
    molid = molecule_ids[0][:, None].astype(jnp.int32)
    modm = jnp.pad(is_molecule_mod[0].astype(f32), ((0, 0), (0, 4)))
    mode = jnp.pad(mod_emb_table, ((0, 4), (0, 0)))
    ws_a, ws_b = W_single[:dtok], W_single[dtok:]
    wpi_a, wpi_b = W_pair_i[:dtok], W_pair_i[dtok:]
    wpj_a, wpj_b = W_pair_j[:dtok], W_pair_j[dtok:]

    amf = additional_molecule_feats[0].astype(jnp.int32)
    biomol = is_molecule_types[0, :, 0:3].any(axis=-1)
    meta = jnp.concatenate(
        [amf, biomol[:, None].astype(jnp.int32),
         jnp.zeros((n, 2), jnp.int32)], axis=1)
    metaT = meta.T
    tbf = token_bonds[0].astype(f32)
    tbfT = tbf.T
    zcol = jnp.zeros((1, dp), f32)
    w_big = jnp.concatenate([
        W_relpos[0:W_RES], jnp.tile(zcol, (72 - W_RES, 1)),
        W_relpos[W_RES:2 * W_RES], jnp.tile(zcol, (72 - W_RES, 1)),
        W_relpos[2 * W_RES:2 * W_RES + 1], jnp.tile(zcol, (7, 1)),
        W_relpos[2 * W_RES + 1:], jnp.tile(zcol, (8 - W_CH, 1)),
    ], axis=0).astype(jnp.bfloat16)

    cst = lambda *_: tuple(0 for _ in range(2))
    single_init, pairwise = pl.pallas_call(
        _body,
        grid=(n // _TI,),
        in_specs=[
            pl.BlockSpec((m, da_in), lambda i: (0, 0)),
            pl.BlockSpec((da_in, da), lambda i: (0, 0)),
            pl.BlockSpec((n, 1), lambda i: (0, 0)),
            pl.BlockSpec((n, dadd), lambda i: (0, 0)),
            pl.BlockSpec((da, dtok), lambda i: (0, 0)),
            pl.BlockSpec((dtok, ds), lambda i: (0, 0)),
            pl.BlockSpec((dadd, ds), lambda i: (0, 0)),
            pl.BlockSpec((n, 1), lambda i: (0, 0)),
            pl.BlockSpec((nmol, ds), lambda i: (0, 0)),
            pl.BlockSpec((n, 8), lambda i: (0, 0)),
            pl.BlockSpec((8, ds), lambda i: (0, 0)),
            pl.BlockSpec((dtok, dp), lambda i: (0, 0)),
            pl.BlockSpec((dadd, dp), lambda i: (0, 0)),
            pl.BlockSpec((dtok, dp), lambda i: (0, 0)),
            pl.BlockSpec((dadd, dp), lambda i: (0, 0)),
            pl.BlockSpec((_TI, 8), lambda i: (i, 0)),
            pl.BlockSpec((8, n), lambda i: (0, 0)),
            pl.BlockSpec((_TI, n), lambda i: (i, 0)),
            pl.BlockSpec((_TI, n), lambda i: (i, 0)),
            pl.BlockSpec((160, dp), lambda i: (0, 0)),
            pl.BlockSpec((1, dp), lambda i: (0, 0)),
        ],
        out_specs=(
            pl.BlockSpec((n, ds), lambda i: (0, 0)),
            pl.BlockSpec((_TI, n, dp), lambda i: (i, 0, 0)),
        ),
        out_shape=(jax.ShapeDtypeStruct((n, ds), f32),
                   jax.ShapeDtypeStruct((n, n, dp), f32)),
        scratch_shapes=[pltpu.VMEM((n, dtok), f32),
                        pltpu.VMEM((n, dp), f32)],
    )(ai, W_atom, lens_col, atf, W_token, ws_a, ws_b, molid, mol_type_emb,
      modm, mode, wpi_a, wpi_b, wpj_a, wpj_b, meta, metaT, tbf, tbfT,
      w_big, w_bond)

    single_mask = (lens_i > 0)[None]
    pairwise_mask = single_mask[:, :, None] & single_mask[:, None, :]
    return single_init[None], single_mask, pairwise[None], pairwise_mask
